# baseline (speedup 1.0000x reference)
import functools

import jax
import jax.numpy as jnp
from jax import lax
from jax.experimental import pallas as pl
from jax.experimental.pallas import tpu as pltpu
from jax.experimental.pallas import tpu_sc as plsc

N_CLS = 8
N_PRE = 5
N_SUF = 64
SEQ = 77
D = 512
HB = 64


def _sc_counts():
    try:
        info = plsc.get_sparse_core_info()
        return int(info.num_cores), int(info.num_subcores)
    except Exception:
        return 2, 16


def kernel(label, cls_ctx, token_prefix, token_suffix):
    B = label.shape[0]
    NC, NS = _sc_counts()
    NW = NC * NS
    bw = B // NW

    lab = label.astype(jnp.int32)
    pref2 = token_prefix.reshape(N_PRE, D)
    cls2 = cls_ctx.reshape(cls_ctx.shape[0] * N_CLS, D)
    suf2 = token_suffix.reshape(token_suffix.shape[0] * N_SUF, D)

    s_col = jnp.arange(SEQ, dtype=jnp.int32)[None, :, None]
    labw = lab.reshape(NW, 1, bw)
    idx = jnp.where(
        s_col < N_PRE, s_col,
        jnp.where(s_col < N_PRE + N_CLS,
                  labw * N_CLS + (s_col - N_PRE),
                  labw * N_SUF + (s_col - (N_PRE + N_CLS))))
    idx_flat = jnp.broadcast_to(idx, (NW, SEQ, bw)).reshape(-1)

    mesh = plsc.VectorSubcoreMesh(core_axis_name="c", subcore_axis_name="s")

    @functools.partial(
        pl.kernel,
        mesh=mesh,
        out_type=jax.ShapeDtypeStruct((SEQ, B, D), jnp.float32),
        scratch_types=[
            pltpu.VMEM((SEQ * bw,), jnp.int32),
            pltpu.VMEM((2, HB, D), jnp.float32),
            pltpu.SemaphoreType.DMA,
            pltpu.SemaphoreType.DMA,
            pltpu.SemaphoreType.DMA,
            pltpu.SemaphoreType.DMA,
        ],
    )
    def _gather(pref_hbm, cls_hbm, suf_hbm, idx_hbm, out_hbm,
                idx_v, buf, g0, g1, w0, w1):
        wid = lax.axis_index("s") * NC + lax.axis_index("c")
        base = wid * bw
        pltpu.sync_copy(idx_hbm.at[pl.ds(base * SEQ, SEQ * bw)], idx_v)
        gsem = (g0, g1)
        wsem = (w0, w1)

        def fire_gather(tab, s, h):
            pltpu.async_copy(
                tab.at[idx_v.at[pl.ds(s * bw + HB * h, HB)]],
                buf.at[h], gsem[h])

        def wait_gather(tab, h):
            pltpu.make_async_copy(
                tab.at[pl.ds(0, HB)], buf.at[h], gsem[h]).wait()

        def fire_write(s, h):
            pltpu.async_copy(
                buf.at[h], out_hbm.at[s, pl.ds(base + HB * h, HB)], wsem[h])

        def wait_write(h):
            pltpu.make_async_copy(
                buf.at[h], out_hbm.at[0, pl.ds(0, HB)], wsem[h]).wait()

        def run_phase(tab, s_lo, s_hi, static):
            fire_gather(tab, s_lo, 0)
            fire_gather(tab, s_lo, 1)

            def step(s, is_last):
                for h in (0, 1):
                    wait_gather(tab, h)
                    fire_write(s, h)
                for h in (0, 1):
                    wait_write(h)
                    if static:
                        if not is_last:
                            fire_gather(tab, s + 1, h)
                    else:
                        @pl.when(jnp.logical_not(is_last))
                        def _():
                            fire_gather(tab, s + 1, h)

            if static:
                for s in range(s_lo, s_hi):
                    step(s, s == s_hi - 1)
            else:
                def body(k, carry):
                    step(s_lo + k, k == s_hi - s_lo - 1)
                    return carry
                lax.fori_loop(0, s_hi - s_lo, body, 0)

        run_phase(pref_hbm, 0, N_PRE, static=True)
        run_phase(cls_hbm, N_PRE, N_PRE + N_CLS, static=True)
        run_phase(suf_hbm, N_PRE + N_CLS, SEQ, static=False)

    res = _gather(pref2, cls2, suf2, idx_flat)
    return jnp.transpose(res, (1, 0, 2))


# speedup vs baseline: 1.2011x; 1.2011x over previous
import functools

import jax
import jax.numpy as jnp
from jax import lax
from jax.experimental import pallas as pl
from jax.experimental.pallas import tpu as pltpu
from jax.experimental.pallas import tpu_sc as plsc

N_CLS = 8
N_PRE = 5
N_SUF = 64
SEQ = 77
SEQ_PAD = 80
D = 512
CPB = 8


def _sc_counts():
    try:
        info = plsc.get_sparse_core_info()
        return int(info.num_cores), int(info.num_subcores)
    except Exception:
        return 2, 16


def _build_fused(token_prefix, cls_ctx, token_suffix):
    n_cls_total = cls_ctx.shape[0]

    def body(p_ref, c_ref, s_ref, o_ref):
        o_ref[:, 0:N_PRE] = jnp.broadcast_to(p_ref[...], (CPB, N_PRE, D))
        o_ref[:, N_PRE:N_PRE + N_CLS] = c_ref[...]
        o_ref[:, N_PRE + N_CLS:SEQ] = s_ref[...]
        o_ref[:, SEQ:SEQ_PAD] = jnp.zeros((CPB, SEQ_PAD - SEQ, D), jnp.float32)

    return pl.pallas_call(
        body,
        grid=(n_cls_total // CPB,),
        in_specs=[
            pl.BlockSpec((1, N_PRE, D), lambda i: (0, 0, 0)),
            pl.BlockSpec((CPB, N_CLS, D), lambda i: (i, 0, 0)),
            pl.BlockSpec((CPB, N_SUF, D), lambda i: (i, 0, 0)),
        ],
        out_specs=pl.BlockSpec((CPB, SEQ_PAD, D), lambda i: (i, 0, 0)),
        out_shape=jax.ShapeDtypeStruct((n_cls_total, SEQ_PAD, D), jnp.float32),
    )(token_prefix, cls_ctx, token_suffix)


def kernel(label, cls_ctx, token_prefix, token_suffix):
    B = label.shape[0]
    NC, NS = _sc_counts()
    NW = NC * NS
    bw = B // NW

    fused = _build_fused(token_prefix, cls_ctx, token_suffix)
    flat = fused.reshape(fused.shape[0] * SEQ_PAD, D)
    lab = label.astype(jnp.int32)
    idx_all = (lab[:, None] * SEQ_PAD
               + jnp.arange(SEQ_PAD, dtype=jnp.int32)).reshape(-1)

    mesh = plsc.VectorSubcoreMesh(core_axis_name="c", subcore_axis_name="s")

    @functools.partial(
        pl.kernel,
        mesh=mesh,
        out_type=jax.ShapeDtypeStruct((B, SEQ, D), jnp.float32),
        scratch_types=[
            pltpu.VMEM((bw * SEQ_PAD,), jnp.int32),
            pltpu.VMEM((2, SEQ_PAD, D), jnp.float32),
            pltpu.SemaphoreType.DMA,
            pltpu.SemaphoreType.DMA,
            pltpu.SemaphoreType.DMA,
            pltpu.SemaphoreType.DMA,
        ],
    )
    def _gather(flat_hbm, idx_hbm, out_hbm, idx_v, buf, g0, g1, w0, w1):
        wid = lax.axis_index("s") * NC + lax.axis_index("c")
        base = wid * bw
        pltpu.sync_copy(idx_hbm.at[pl.ds(base * SEQ_PAD, bw * SEQ_PAD)], idx_v)
        gsem = (g0, g1)
        wsem = (w0, w1)

        def fire_gather(j, h):
            pltpu.async_copy(
                flat_hbm.at[idx_v.at[pl.ds(j * SEQ_PAD, SEQ_PAD)]],
                buf.at[h], gsem[h])

        fire_gather(0, 0)
        fire_gather(1, 1)

        def body(k, carry):
            for h in (0, 1):
                j = 2 * k + h
                pltpu.make_async_copy(
                    flat_hbm.at[pl.ds(0, SEQ_PAD)], buf.at[h], gsem[h]).wait()
                pltpu.async_copy(
                    buf.at[h], out_hbm.at[base + j, pl.ds(0, SEQ_PAD)],
                    wsem[h])
            for h in (0, 1):
                pltpu.make_async_copy(
                    buf.at[h], out_hbm.at[0, pl.ds(0, SEQ_PAD)],
                    wsem[h]).wait()

                @pl.when(k < bw // 2 - 1)
                def _():
                    fire_gather(2 * k + h + 2, h)
            return carry

        lax.fori_loop(0, bw // 2, body, 0)

    return _gather(flat, idx_all)
